# baseline (speedup 1.0000x reference)
import functools

import jax
import jax.numpy as jnp
from jax import lax
from jax.experimental import pallas as pl
from jax.experimental.pallas import tpu as pltpu
from jax.experimental.pallas import tpu_sc as plsc

L = 16
NC = 2
NS = 16
NW = NC * NS

B = 16384
D = 32
CHUNK = B // NW
NSTREAM = 4
IDXW = CHUNK // NSTREAM


def _mf_body(users_hbm, items_hbm, utab_hbm, itab_hbm, out_hbm,
             uidx_v, iidx_v, ubuf, ibuf, out_v, sem):
    wid = lax.axis_index("s") * NC + lax.axis_index("c")
    base = wid * CHUNK

    pltpu.sync_copy(users_hbm.at[wid], uidx_v)
    pltpu.sync_copy(items_hbm.at[wid], iidx_v)

    for c in range(D):
        copies = []
        for j in range(NSTREAM):
            copies.append(pltpu.async_copy(
                utab_hbm.at[c].at[uidx_v.at[j]],
                ubuf.at[c].at[pl.ds(j * IDXW, IDXW)], sem))
            copies.append(pltpu.async_copy(
                itab_hbm.at[c].at[iidx_v.at[j]],
                ibuf.at[c].at[pl.ds(j * IDXW, IDXW)], sem))
        for cp in copies:
            cp.wait()

    def group(g, carry):
        s = pl.ds(g * L, L)
        acc = jnp.zeros((L,), jnp.float32)
        for c in range(D):
            acc = acc + ubuf[c, s] * ibuf[c, s]
        out_v[s] = acc
        return carry

    lax.fori_loop(0, CHUNK // L, group, 0)

    pltpu.sync_copy(out_v, out_hbm.at[pl.ds(base, CHUNK)])


@functools.partial(
    pl.kernel,
    out_type=jax.ShapeDtypeStruct((B,), jnp.float32),
    mesh=plsc.VectorSubcoreMesh(core_axis_name="c", subcore_axis_name="s"),
    scratch_types=[
        pltpu.VMEM((NSTREAM, IDXW), jnp.int32),
        pltpu.VMEM((NSTREAM, IDXW), jnp.int32),
        pltpu.VMEM((D, CHUNK), jnp.float32),
        pltpu.VMEM((D, CHUNK), jnp.float32),
        pltpu.VMEM((CHUNK,), jnp.float32),
        pltpu.SemaphoreType.DMA,
    ],
    compiler_params=pltpu.CompilerParams(
        needs_layout_passes=False, use_tc_tiling_on_sc=False),
)
def _mf(users_hbm, items_hbm, utab_hbm, itab_hbm, out_hbm,
        uidx_v, iidx_v, ubuf, ibuf, out_v, sem):
    _mf_body(users_hbm, items_hbm, utab_hbm, itab_hbm, out_hbm,
             uidx_v, iidx_v, ubuf, ibuf, out_v, sem)


def kernel(users, items, user_table, item_table):
    u = users.astype(jnp.int32).reshape(NW, NSTREAM, IDXW)
    it = items.astype(jnp.int32).reshape(NW, NSTREAM, IDXW)
    return _mf(u, it, user_table.T, item_table.T)


# speedup vs baseline: 5.6674x; 5.6674x over previous
import functools

import jax
import jax.numpy as jnp
from jax import lax
from jax.experimental import pallas as pl
from jax.experimental.pallas import tpu as pltpu
from jax.experimental.pallas import tpu_sc as plsc

L = 16
NC = 2
NS = 16
NW = NC * NS

B = 16384
D = 32
CHUNK = B // NW
NSTREAM = 4
IDXW = CHUNK // NSTREAM


def _mf_body(users_hbm, items_hbm, user_table_hbm, item_table_hbm, out_hbm,
             uidx_v, iidx_v, urows_v, irows_v, out_v, sem):
    wid = lax.axis_index("s") * NC + lax.axis_index("c")
    base = wid * CHUNK

    pltpu.sync_copy(users_hbm.at[wid], uidx_v)
    pltpu.sync_copy(items_hbm.at[wid], iidx_v)

    copies = []
    for j in range(NSTREAM):
        copies.append(pltpu.async_copy(
            user_table_hbm.at[uidx_v.at[j]],
            urows_v.at[pl.ds(j * IDXW, IDXW)], sem))
        copies.append(pltpu.async_copy(
            item_table_hbm.at[iidx_v.at[j]],
            irows_v.at[pl.ds(j * IDXW, IDXW)], sem))
    for c in copies:
        c.wait()

    def group(g, carry):
        row = g * L + lax.iota(jnp.int32, L)
        acc = jnp.zeros((L,), jnp.float32)
        for d in range(D):
            col = jnp.full((L,), d, jnp.int32)
            u = plsc.load_gather(urows_v, [row, col])
            v = plsc.load_gather(irows_v, [row, col])
            acc = acc + u * v
        out_v[pl.ds(g * L, L)] = acc
        return carry

    lax.fori_loop(0, CHUNK // L, group, 0)

    pltpu.sync_copy(out_v, out_hbm.at[pl.ds(base, CHUNK)])


@functools.partial(
    pl.kernel,
    out_type=jax.ShapeDtypeStruct((B,), jnp.float32),
    mesh=plsc.VectorSubcoreMesh(core_axis_name="c", subcore_axis_name="s"),
    scratch_types=[
        pltpu.VMEM((NSTREAM, IDXW), jnp.int32),
        pltpu.VMEM((NSTREAM, IDXW), jnp.int32),
        pltpu.VMEM((CHUNK, D), jnp.float32),
        pltpu.VMEM((CHUNK, D), jnp.float32),
        pltpu.VMEM((CHUNK,), jnp.float32),
        pltpu.SemaphoreType.DMA,
    ],
    compiler_params=pltpu.CompilerParams(
        needs_layout_passes=False, use_tc_tiling_on_sc=False),
)
def _mf(users_hbm, items_hbm, user_table_hbm, item_table_hbm, out_hbm,
        uidx_v, iidx_v, urows_v, irows_v, out_v, sem):
    _mf_body(users_hbm, items_hbm, user_table_hbm, item_table_hbm, out_hbm,
             uidx_v, iidx_v, urows_v, irows_v, out_v, sem)


def kernel(users, items, user_table, item_table):
    u = users.astype(jnp.int32).reshape(NW, NSTREAM, IDXW)
    it = items.astype(jnp.int32).reshape(NW, NSTREAM, IDXW)
    return _mf(u, it, user_table, item_table)


# speedup vs baseline: 18.9878x; 3.3504x over previous
import functools

import jax
import jax.numpy as jnp
from jax import lax
from jax.experimental import pallas as pl
from jax.experimental.pallas import tpu as pltpu
from jax.experimental.pallas import tpu_sc as plsc

L = 16
NC = 2
NS = 16
NW = NC * NS

B = 16384
D = 32
V = 1_000_000
VTAIL = V - (V // 128) * 128 + 64
TAILLO = 999936
SHARD = 31744
STRIDE = 31232
BLK = 1024
NBLK = SHARD // BLK
CHUNK = B // NW
SROWS = B + 8 * NW
CAP = B + L


def _stage_body(users_hbm, items_hbm, tab_hbms, tail_hbms, st_hbms,
                idx_v, bidx_v, bpos_v, cidx_v, cpos_v,
                blk_v, tail_v, slab_v, sem):
    wid = lax.axis_index("s") * NC + lax.axis_index("c")
    lo = wid * STRIDE
    lanes = lax.iota(jnp.int32, L)
    dummy = B + wid * 8 + (lanes & 7)
    is31 = wid == NW - 1

    for t in range(2):
        src_idx = (users_hbm, items_hbm)[t]
        tab = tab_hbms[t]
        st = st_hbms[t]
        pltpu.sync_copy(src_idx, idx_v)
        pltpu.sync_copy(tail_hbms[t], tail_v)

        def scan(g, n):
            idx16 = idx_v[pl.ds(g * L, L)]
            m = (idx16 >= lo) & (idx16 < lo + SHARD)
            m = m | ((idx16 >= TAILLO) & is31)
            plsc.store_compressed(bidx_v.at[pl.ds(n, L)], idx16, mask=m)
            plsc.store_compressed(bpos_v.at[pl.ds(n, L)], g * L + lanes, mask=m)
            return n + plsc.all_reduce_population_count(m)[0]

        n = lax.fori_loop(0, B // L, scan, 0)
        qmax = (n + L - 1) // L

        def extract(src_ref, blo, nb):
            def emit(e, carry):
                valid = e * L + lanes < nb
                col = jnp.where(valid, cidx_v[pl.ds(e * L, L)], 0)
                pos = jnp.where(valid, cpos_v[pl.ds(e * L, L)], dummy)
                for c in range(D):
                    cc = jnp.full((L,), c, jnp.int32)
                    if src_ref is blk_v:
                        val = plsc.load_gather(src_ref, [cc, col])
                    else:
                        val = plsc.load_gather(src_ref, [col, cc])
                    plsc.store_scatter(slab_v, [lanes, cc], val)
                pltpu.async_copy(slab_v, st.at[pos], sem).wait()
                return carry
            del blo
            lax.fori_loop(0, (nb + L - 1) // L, emit, 0)

        def filt(blo, bhi):
            def fscan(q, nb):
                valid = q * L + lanes < n
                bi = bidx_v[pl.ds(q * L, L)]
                bp = bpos_v[pl.ds(q * L, L)]
                m = valid & (bi >= blo) & (bi < bhi)
                plsc.store_compressed(cidx_v.at[pl.ds(nb, L)], bi - blo, mask=m)
                plsc.store_compressed(cpos_v.at[pl.ds(nb, L)], bp, mask=m)
                return nb + plsc.all_reduce_population_count(m)[0]
            return lax.fori_loop(0, qmax, fscan, 0)

        def block(b, carry):
            blo = lo + b * BLK
            pltpu.sync_copy(tab.at[:, pl.ds(blo, BLK)], blk_v)
            nb = filt(blo, blo + BLK)
            extract(blk_v, blo, nb)
            return carry

        lax.fori_loop(0, NBLK, block, 0)

        @pl.when(is31)
        def _():
            nb = filt(TAILLO, V)
            extract(tail_v, TAILLO, nb)


@functools.partial(
    pl.kernel,
    out_type=(jax.ShapeDtypeStruct((SROWS, 128), jnp.float32),
              jax.ShapeDtypeStruct((SROWS, 128), jnp.float32)),
    mesh=plsc.VectorSubcoreMesh(core_axis_name="c", subcore_axis_name="s"),
    scratch_types=[
        pltpu.VMEM((B,), jnp.int32),
        pltpu.VMEM((CAP,), jnp.int32),
        pltpu.VMEM((CAP,), jnp.int32),
        pltpu.VMEM((CAP,), jnp.int32),
        pltpu.VMEM((CAP,), jnp.int32),
        pltpu.VMEM((D, BLK), jnp.float32),
        pltpu.VMEM((64, D), jnp.float32),
        pltpu.VMEM((L, 128), jnp.float32),
        pltpu.SemaphoreType.DMA,
    ],
    compiler_params=pltpu.CompilerParams(
        needs_layout_passes=False, use_tc_tiling_on_sc=True),
)
def _stage(users_hbm, items_hbm, utab, itab, utail, itail,
           st_u, st_i,
           idx_v, bidx_v, bpos_v, cidx_v, cpos_v, blk_v, tail_v, slab_v,
           sem):
    _stage_body(users_hbm, items_hbm, (utab, itab), (utail, itail),
                (st_u, st_i),
                idx_v, bidx_v, bpos_v, cidx_v, cpos_v,
                blk_v, tail_v, slab_v, sem)


def _dot_body(st_u, st_i, out_hbm, ubuf, ibuf, out_v, sem):
    wid = lax.axis_index("s") * NC + lax.axis_index("c")
    base = wid * CHUNK

    for h in range(2):
        hb = base + h * (CHUNK // 2)
        pltpu.sync_copy(st_u.at[pl.ds(hb, CHUNK // 2)], ubuf)
        pltpu.sync_copy(st_i.at[pl.ds(hb, CHUNK // 2)], ibuf)

        def group(g, carry):
            row = g * L + lax.iota(jnp.int32, L)
            acc = jnp.zeros((L,), jnp.float32)
            for d in range(D):
                col = jnp.full((L,), d, jnp.int32)
                u = plsc.load_gather(ubuf, [row, col])
                v = plsc.load_gather(ibuf, [row, col])
                acc = acc + u * v
            out_v[pl.ds(h * (CHUNK // 2) + g * L, L)] = acc
            return carry

        lax.fori_loop(0, CHUNK // 2 // L, group, 0)

    pltpu.sync_copy(out_v, out_hbm.at[pl.ds(base, CHUNK)])


@functools.partial(
    pl.kernel,
    out_type=jax.ShapeDtypeStruct((B,), jnp.float32),
    mesh=plsc.VectorSubcoreMesh(core_axis_name="c", subcore_axis_name="s"),
    scratch_types=[
        pltpu.VMEM((CHUNK // 2, 128), jnp.float32),
        pltpu.VMEM((CHUNK // 2, 128), jnp.float32),
        pltpu.VMEM((CHUNK,), jnp.float32),
        pltpu.SemaphoreType.DMA,
    ],
    compiler_params=pltpu.CompilerParams(
        needs_layout_passes=False, use_tc_tiling_on_sc=True),
)
def _dot(st_u, st_i, out_hbm, ubuf, ibuf, out_v, sem):
    _dot_body(st_u, st_i, out_hbm, ubuf, ibuf, out_v, sem)


def kernel(users, items, user_table, item_table):
    u = users.astype(jnp.int32)
    it = items.astype(jnp.int32)
    ut = user_table.T
    itb = item_table.T
    utail = user_table[TAILLO:]
    itail = item_table[TAILLO:]
    st_u, st_i = _stage(u, it, ut, itb, utail, itail)
    return _dot(st_u, st_i)


# speedup vs baseline: 26.1318x; 1.3762x over previous
import functools

import jax
import jax.numpy as jnp
from jax import lax
from jax.experimental import pallas as pl
from jax.experimental.pallas import tpu as pltpu
from jax.experimental.pallas import tpu_sc as plsc

L = 16
NC = 2
NS = 16
NW = NC * NS

B = 16384
D = 32
V = 1_000_000
VTAIL = V - (V // 128) * 128 + 64
TAILLO = 999936
SHARD = 31744
STRIDE = 31232
BLK = 1024
NBLK = SHARD // BLK
CHUNK = B // NW
SROWS = B + 8 * NW
CAP = B + L


def _stage_body(users_hbm, items_hbm, tab_hbms, tail_hbms, st_hbms,
                idx_v, bkt_v, cl_v,
                blk_a, blk_b, tail_v, slab_v, sem_a, sem_b, sem_c):
    wid = lax.axis_index("s") * NC + lax.axis_index("c")
    lo = wid * STRIDE
    lanes = lax.iota(jnp.int32, L)
    dummy = B + wid * 8 + (lanes & 7)
    is31 = wid == NW - 1

    for t in range(2):
        src_idx = (users_hbm, items_hbm)[t]
        tab = tab_hbms[t]
        st = st_hbms[t]
        pltpu.sync_copy(src_idx, idx_v)
        pltpu.sync_copy(tail_hbms[t], tail_v)

        def scan(g, n):
            li = idx_v[pl.ds(g * L, L)] - lo
            m = (li >= 0) & (li < SHARD)
            m = m | ((li >= SHARD) & (li < SHARD + 64) & is31)
            packed = (li << 14) | (g * L + lanes)
            plsc.store_compressed(bkt_v.at[pl.ds(n, L)], packed, mask=m)
            return n + plsc.all_reduce_population_count(m)[0]

        n = lax.fori_loop(0, B // L, scan, 0)
        qmax = (n + L - 1) // L

        def extract(src_ref, blo_rel, nb):
            def emit(e, carry):
                valid = e * L + lanes < nb
                packed = cl_v[pl.ds(e * L, L)]
                col = jnp.where(valid, (packed >> 14) - blo_rel, 0)
                pos = jnp.where(valid, packed & (B - 1), dummy)
                for c in range(D):
                    cc = jnp.full((L,), c, jnp.int32)
                    if src_ref is tail_v:
                        val = plsc.load_gather(src_ref, [col, cc])
                    else:
                        val = plsc.load_gather(src_ref, [cc, col])
                    plsc.store_scatter(slab_v, [lanes, cc], val)
                pltpu.async_copy(slab_v, st.at[pos], sem_c).wait()
                return carry
            lax.fori_loop(0, (nb + L - 1) // L, emit, 0)

        def filt(blo_rel, bhi_rel):
            def fscan(q, nb):
                valid = q * L + lanes < n
                packed = bkt_v[pl.ds(q * L, L)]
                li = packed >> 14
                m = valid & (li >= blo_rel) & (li < bhi_rel)
                plsc.store_compressed(cl_v.at[pl.ds(nb, L)], packed, mask=m)
                return nb + plsc.all_reduce_population_count(m)[0]
            return lax.fori_loop(0, qmax, fscan, 0)

        def fire(b, buf, sem):
            return pltpu.async_copy(tab.at[:, pl.ds(lo + b * BLK, BLK)],
                                    buf, sem)

        def drain(buf, sem):
            pltpu.make_async_copy(tab.at[:, pl.ds(0, BLK)], buf, sem).wait()

        def process(src_ref, b_rel):
            nb = filt(b_rel, b_rel + BLK)
            extract(src_ref, b_rel, nb)

        fire(0, blk_a, sem_a)

        def pair(p, carry):
            fire(2 * p + 1, blk_b, sem_b)
            drain(blk_a, sem_a)
            process(blk_a, 2 * p * BLK)

            @pl.when(p < NBLK // 2 - 1)
            def _():
                fire(2 * p + 2, blk_a, sem_a)

            @pl.when(p == NBLK // 2 - 1)
            def _():
                fire(NBLK - 1, blk_a, sem_a)

            drain(blk_b, sem_b)
            process(blk_b, (2 * p + 1) * BLK)
            return carry

        lax.fori_loop(0, NBLK // 2, pair, 0)
        drain(blk_a, sem_a)
        process(blk_a, (NBLK - 1) * BLK)

        @pl.when(is31)
        def _():
            nb = filt(SHARD, SHARD + 64)
            extract(tail_v, SHARD, nb)


@functools.partial(
    pl.kernel,
    out_type=(jax.ShapeDtypeStruct((SROWS, 128), jnp.float32),
              jax.ShapeDtypeStruct((SROWS, 128), jnp.float32)),
    mesh=plsc.VectorSubcoreMesh(core_axis_name="c", subcore_axis_name="s"),
    scratch_types=[
        pltpu.VMEM((B,), jnp.int32),
        pltpu.VMEM((CAP,), jnp.int32),
        pltpu.VMEM((CAP,), jnp.int32),
        pltpu.VMEM((D, BLK), jnp.float32),
        pltpu.VMEM((D, BLK), jnp.float32),
        pltpu.VMEM((64, D), jnp.float32),
        pltpu.VMEM((L, 128), jnp.float32),
        pltpu.SemaphoreType.DMA,
        pltpu.SemaphoreType.DMA,
        pltpu.SemaphoreType.DMA,
    ],
    compiler_params=pltpu.CompilerParams(
        needs_layout_passes=False, use_tc_tiling_on_sc=True),
)
def _stage(users_hbm, items_hbm, utab, itab, utail, itail,
           st_u, st_i,
           idx_v, bkt_v, cl_v, blk_a, blk_b, tail_v, slab_v,
           sem_a, sem_b, sem_c):
    _stage_body(users_hbm, items_hbm, (utab, itab), (utail, itail),
                (st_u, st_i),
                idx_v, bkt_v, cl_v,
                blk_a, blk_b, tail_v, slab_v, sem_a, sem_b, sem_c)


def _dot_body(st_u, st_i, out_hbm, ubuf, ibuf, out_v, sem):
    wid = lax.axis_index("s") * NC + lax.axis_index("c")
    base = wid * CHUNK

    for h in range(2):
        hb = base + h * (CHUNK // 2)
        pltpu.sync_copy(st_u.at[pl.ds(hb, CHUNK // 2)], ubuf)
        pltpu.sync_copy(st_i.at[pl.ds(hb, CHUNK // 2)], ibuf)

        def group(g, carry):
            row = g * L + lax.iota(jnp.int32, L)
            acc = jnp.zeros((L,), jnp.float32)
            for d in range(D):
                col = jnp.full((L,), d, jnp.int32)
                u = plsc.load_gather(ubuf, [row, col])
                v = plsc.load_gather(ibuf, [row, col])
                acc = acc + u * v
            out_v[pl.ds(h * (CHUNK // 2) + g * L, L)] = acc
            return carry

        lax.fori_loop(0, CHUNK // 2 // L, group, 0)

    pltpu.sync_copy(out_v, out_hbm.at[pl.ds(base, CHUNK)])


@functools.partial(
    pl.kernel,
    out_type=jax.ShapeDtypeStruct((B,), jnp.float32),
    mesh=plsc.VectorSubcoreMesh(core_axis_name="c", subcore_axis_name="s"),
    scratch_types=[
        pltpu.VMEM((CHUNK // 2, 128), jnp.float32),
        pltpu.VMEM((CHUNK // 2, 128), jnp.float32),
        pltpu.VMEM((CHUNK,), jnp.float32),
        pltpu.SemaphoreType.DMA,
    ],
    compiler_params=pltpu.CompilerParams(
        needs_layout_passes=False, use_tc_tiling_on_sc=True),
)
def _dot(st_u, st_i, out_hbm, ubuf, ibuf, out_v, sem):
    _dot_body(st_u, st_i, out_hbm, ubuf, ibuf, out_v, sem)


def kernel(users, items, user_table, item_table):
    u = users.astype(jnp.int32)
    it = items.astype(jnp.int32)
    ut = user_table.T
    itb = item_table.T
    utail = user_table[TAILLO:]
    itail = item_table[TAILLO:]
    st_u, st_i = _stage(u, it, ut, itb, utail, itail)
    return _dot(st_u, st_i)
